# baseline (speedup 1.0000x reference)
import functools

import jax
import jax.numpy as jnp
from jax import lax
from jax.experimental import pallas as pl
from jax.experimental.pallas import tpu as pltpu
from jax.experimental.pallas import tpu_sc as plsc

_B = 16384
_D = 64
_NC = 2
_NS = 16
_NW = _NC * _NS
_CHUNK = _B // _NW
_G = 16
_NGROUPS = _CHUNK // _G


def _als_body(uid_hbm, iid_hbm, u_hbm, i_hbm, ub_hbm, ib_hbm, out_hbm,
              uid_v, iid_v, urows_v, irows_v, ub_v, ib_v, out_v, sem):
    wid = lax.axis_index("s") * _NC + lax.axis_index("c")
    base = wid * _CHUNK

    pltpu.sync_copy(uid_hbm.at[pl.ds(base, _CHUNK)], uid_v)
    pltpu.sync_copy(iid_hbm.at[pl.ds(base, _CHUNK)], iid_v)

    cp_u = pltpu.async_copy(u_hbm.at[uid_v], urows_v, sem)
    cp_i = pltpu.async_copy(i_hbm.at[iid_v], irows_v, sem)
    cp_ub = pltpu.async_copy(ub_hbm.at[uid_v], ub_v, sem)
    cp_ib = pltpu.async_copy(ib_hbm.at[iid_v], ib_v, sem)
    cp_u.wait()
    cp_i.wait()
    cp_ub.wait()
    cp_ib.wait()

    lanes = lax.iota(jnp.int32, 16)

    def group(g, carry):
        tot = jnp.zeros((16,), jnp.float32)
        for r in range(_G):
            row = g * _G + r
            acc = urows_v[row, pl.ds(0, 16)] * irows_v[row, pl.ds(0, 16)]
            for c in range(1, _D // 16):
                acc = acc + (urows_v[row, pl.ds(c * 16, 16)]
                             * irows_v[row, pl.ds(c * 16, 16)])
            tot = jnp.where(lanes == r, jnp.sum(acc), tot)
        tot = tot + ub_v[pl.ds(g * 16, 16)] + ib_v[pl.ds(g * 16, 16)]
        out_v[pl.ds(g * 16, 16)] = tot
        return carry

    lax.fori_loop(0, _NGROUPS, group, 0)

    pltpu.sync_copy(out_v, out_hbm.at[pl.ds(base, _CHUNK)])


_als = functools.partial(
    pl.kernel,
    out_type=jax.ShapeDtypeStruct((_B,), jnp.float32),
    mesh=plsc.VectorSubcoreMesh(core_axis_name="c", subcore_axis_name="s",
                                num_cores=_NC, num_subcores=_NS),
    compiler_params=pltpu.CompilerParams(needs_layout_passes=False,
                                         use_tc_tiling_on_sc=False),
    scratch_types=[
        pltpu.VMEM((_CHUNK,), jnp.int32),
        pltpu.VMEM((_CHUNK,), jnp.int32),
        pltpu.VMEM((_CHUNK, _D), jnp.float32),
        pltpu.VMEM((_CHUNK, _D), jnp.float32),
        pltpu.VMEM((_CHUNK,), jnp.float32),
        pltpu.VMEM((_CHUNK,), jnp.float32),
        pltpu.VMEM((_CHUNK,), jnp.float32),
        pltpu.SemaphoreType.DMA,
    ],
)(_als_body)


def kernel(user_id, item_id, u, i, u_bias, i_bias):
    return _als(user_id.astype(jnp.int32), item_id.astype(jnp.int32),
                u, i, u_bias, i_bias)
